# Initial kernel scaffold; baseline (speedup 1.0000x reference)
import jax
import jax.numpy as jnp
from jax.experimental import pallas as pl


def kernel(adj, edges_features, nodes_features, W_mu, b_mu, W_ls, b_ls, W_d1, b_d1, W_d2, b_d2, W_nd, b_nd, W_ed, b_ed, eps):
    raise NotImplementedError("write your pallas kernel here")


# speedup vs baseline: 675.5595x; 675.5595x over previous
import functools

import numpy as np
import jax
import jax.numpy as jnp
from jax import lax
from jax.experimental import pallas as pl

N = 9
EM = 36
NFD = 11
LAT = 128
HH = N * NFD
ODIM = N * (N + 1) // 2
NEG = -1e30
F32 = jnp.float32


def _roll_lanes(g, k):
    return jnp.concatenate([g[:, -k:], g[:, :-k]], axis=1)


def _body(adj_ref, ef_ref, nf_ref, Wmu_ref, bmu_ref, Wls_ref, bls_ref,
          Wd1_ref, bd1_ref, Wd2_ref, bd2_ref, Wnd_ref, bnd_ref,
          Wed_ref, bed_ref, eps_ref, out_ref):
    adj = adj_ref[...]
    ef_all = ef_ref[...]
    gh = nf_ref[...]
    eps = eps_ref[...]

    dot = functools.partial(jnp.dot, preferred_element_type=jnp.float32)
    z_mu = dot(gh, Wmu_ref[...]) + bmu_ref[...]
    z_ls = dot(gh, Wls_ref[...]) + bls_ref[...]
    z = z_mu + eps * jnp.exp(0.5 * z_ls)
    y = jnp.maximum(dot(z, Wd1_ref[...]) + bd1_ref[...], 0.0)
    hdec = dot(y, Wd2_ref[...]) + bd2_ref[...]
    out = jax.nn.sigmoid(hdec)
    node_recon = dot(y, Wnd_ref[...]) + bnd_ref[...]
    ed144 = dot(y, Wed_ref[...]) + bed_ref[...]
    e_r = lax.broadcasted_iota(jnp.int32, (EM, 4 * EM), 0)
    e_c = lax.broadcasted_iota(jnp.int32, (EM, 4 * EM), 1)
    Asel = ((e_c // 4) == e_r).astype(F32)
    b_r = lax.broadcasted_iota(jnp.int32, (4 * EM, 4), 0)
    b_c = lax.broadcasted_iota(jnp.int32, (4 * EM, 4), 1)
    Bsel = ((b_r % 4) == b_c).astype(F32)
    ed_logits = dot(Asel * ed144, Bsel)

    edm = jnp.max(ed_logits, axis=1, keepdims=True)
    ede = jnp.exp(ed_logits - edm)
    er = ede / jnp.sum(ede, axis=1, keepdims=True)

    rows = []
    base = 0
    for r in range(N):
        seg = out[:, base:base + (N - r)]
        if r > 0:
            seg = jnp.concatenate([jnp.zeros((1, r), F32), seg], axis=1)
        rows.append(seg)
        base += N - r
    low = jnp.concatenate(rows, axis=0)

    r9 = lax.broadcasted_iota(jnp.int32, (N, N), 0)
    c9 = lax.broadcasted_iota(jnp.int32, (N, N), 1)
    eyeM = (r9 == c9).astype(F32)
    triuM = (r9 <= c9).astype(F32)

    def _tr(m):
        return lax.dot_general(eyeM, m, (((1,), (1,)), ((), ())),
                               preferred_element_type=jnp.float32)

    lowT = _tr(low)
    adjr = low + lowT - low * eyeM

    adjT = _tr(adj)
    aw_col = jnp.concatenate(
        [adjT[r + 1:N, r:r + 1] for r in range(N - 1)], axis=0)
    edges_total = er * aw_col

    ef9 = ef_all[:N, :]
    efr9 = er[:N, :]
    outer = functools.partial(
        lax.dot_general, dimension_numbers=(((1,), (1,)), ((), ())),
        preferred_element_type=jnp.float32)
    dots = outer(ef9, efr9)
    n1 = jnp.sqrt(jnp.sum(ef9 * ef9, axis=1, keepdims=True))
    n2 = jnp.sqrt(jnp.sum(efr9 * efr9, axis=1, keepdims=True))
    denom = jnp.maximum(outer(n1, n2), 1e-8)
    cosm = dots / denom

    dadj = jnp.sum(adj * eyeM, axis=1, keepdims=True)
    dadjr = jnp.sum(adjr * eyeM, axis=1, keepdims=True)
    diag_term = outer(dadj, dadjr) * cosm

    f_r = lax.broadcasted_iota(jnp.int32, (N * N, N), 0)
    f_c = lax.broadcasted_iota(jnp.int32, (N * N, N), 1)
    RowSel = ((f_r // N) == f_c).astype(F32)
    ModMsk = ((f_r % N) == f_c).astype(F32)
    adj_col = jnp.sum(dot(RowSel, adj) * ModMsk,
                      axis=1, keepdims=True)

    g_r = lax.broadcasted_iota(jnp.int32, (N, N * N), 0)
    g_c = lax.broadcasted_iota(jnp.int32, (N, N * N), 1)
    ColSel = (g_r == (g_c % N)).astype(F32)
    DivMsk = (g_r == (g_c // N)).astype(F32)
    adjr_row = jnp.sum(dot(adjr, ColSel) * DivMsk,
                       axis=0, keepdims=True)
    base_S = jnp.abs(adj_col - adjr_row)

    vR = lax.broadcasted_iota(jnp.int32, (N * N, 1), 0)
    vC = lax.broadcasted_iota(jnp.int32, (1, N * N), 1)
    eyeR = (vR // N) == (vR % N)
    eyeC = (vC // N) == (vC % N)
    offmask = ((~eyeR) & (~eyeC)).astype(F32)

    dt_c = jnp.concatenate([diag_term] * N, axis=1)
    dt_tile = jnp.concatenate([dt_c] * N, axis=0)
    S = jnp.where(eyeR & eyeC, dt_tile, base_S * offmask)

    rm_r = lax.broadcasted_iota(jnp.int32, (N, N * N), 0)
    rm_c = lax.broadcasted_iota(jnp.int32, (N, N * N), 1)
    Rm = (((rm_c // N) == rm_r) & ((rm_c % N) != rm_r)).astype(F32)

    def mpm_step(_, x):
        xc = jnp.concatenate([x] * N, axis=1)
        xr = jnp.concatenate([xc] * N, axis=0)
        prod = S * xr
        pmax = jnp.concatenate(
            [jnp.max(prod[:, a * N:(a + 1) * N], axis=1, keepdims=True)
             for a in range(N)], axis=1)
        neigh = dot(Rm, pmax)
        x_new = x * diag_term + neigh
        nrm = jnp.sqrt(jnp.sum(x_new * x_new))
        return x_new / nrm

    x0 = jnp.full((N, N), 1.0 / N, F32)
    x = lax.fori_loop(0, 50, mpm_step, x0)

    iota512 = lax.broadcasted_iota(jnp.int32, (1, 512), 1)
    bitmask = [((iota512 >> j) & 1) == 1 for j in range(N)]

    g = jnp.zeros((1, 512), F32)
    G = [g]
    for i in range(N - 1, -1, -1):
        cands = []
        for j in range(N):
            shifted = _roll_lanes(g, 1 << j)
            xij = x[i:i + 1, j:j + 1]
            cands.append(jnp.where(bitmask[j], shifted + xij, NEG))
        m = cands[0]
        for cnd in cands[1:]:
            m = jnp.maximum(m, cnd)
        g = m
        G.append(g)

    iota9r = lax.broadcasted_iota(jnp.int32, (1, N), 1)
    pow2r = jnp.left_shift(jnp.int32(1), iota9r)

    s_cur = jnp.int32(511)
    avail = jnp.ones((1, N), jnp.bool_)
    p_rows = []
    for i in range(N):
        g_i = G[N - i]
        g_n = G[N - i - 1]
        cur_val = jnp.sum(jnp.where(iota512 == s_cur, g_i, 0.0))
        vals = jnp.concatenate(
            [jnp.broadcast_to(
                jnp.sum(jnp.where(iota512 == (s_cur - (1 << j)), g_n, 0.0)),
                (1, 1))
             for j in range(N)], axis=1)
        elig = avail & ((x[i:i + 1, :] + vals) == cur_val)
        j_pick = jnp.min(jnp.where(elig, iota9r, 9))
        p_rows.append((iota9r == j_pick).astype(F32))
        avail = avail & (iota9r != j_pick)
        s_cur = s_cur - jnp.sum(jnp.where(iota9r == j_pick, pow2r, 0))

    P = jnp.concatenate(p_rows, axis=0)
    PtA = lax.dot_general(P, adj, (((0,), (0,)), ((), ())),
                          preferred_element_type=jnp.float32)
    adj_perm = dot(PtA, P)

    pclip = jnp.clip(low, 1e-7, 1.0 - 1e-7)
    bce_mat = adj_perm * jnp.log(pclip) + (1.0 - adj_perm) * jnp.log(1.0 - pclip)
    adj_recon_loss = -jnp.sum(bce_mat * triuM) / ODIM

    loss_kl = -0.5 * jnp.sum(1.0 + z_ls - z_mu * z_mu - jnp.exp(z_ls)) / (N * N)
    diff_e = edges_total - ef_all
    loss_edge = jnp.sum(diff_e * diff_e) / (EM * 4)
    diff_n = node_recon - gh
    loss_node = jnp.sum(diff_n * diff_n) / HH

    total = adj_recon_loss + loss_kl + loss_edge + loss_node
    out_ref[...] = jnp.broadcast_to(total, (1, 1))


def kernel(adj, edges_features, nodes_features, W_mu, b_mu, W_ls, b_ls,
           W_d1, b_d1, W_d2, b_d2, W_nd, b_nd, W_ed, b_ed, eps):
    adj0 = adj[0]
    ef = edges_features[0]
    gh = nodes_features.reshape(1, HH)
    res = pl.pallas_call(
        _body,
        out_shape=jax.ShapeDtypeStruct((1, 1), jnp.float32),
    )(adj0, ef, gh,
      W_mu, b_mu.reshape(1, -1), W_ls, b_ls.reshape(1, -1),
      W_d1, b_d1.reshape(1, -1), W_d2, b_d2.reshape(1, -1),
      W_nd, b_nd.reshape(1, -1), W_ed, b_ed.reshape(1, -1),
      eps.reshape(1, -1))
    return res[0, 0]


# speedup vs baseline: 731.5073x; 1.0828x over previous
import functools

import numpy as np
import jax
import jax.numpy as jnp
from jax import lax
from jax.experimental import pallas as pl

N = 9
EM = 36
NFD = 11
LAT = 128
HH = N * NFD
ODIM = N * (N + 1) // 2
NEG = -1e30
F32 = jnp.float32


def _roll_lanes(g, k):
    return jnp.concatenate([g[:, -k:], g[:, :-k]], axis=1)


def _body(adj_ref, ef_ref, nf_ref, Wmu_ref, bmu_ref, Wls_ref, bls_ref,
          Wd1_ref, bd1_ref, Wd2_ref, bd2_ref, Wnd_ref, bnd_ref,
          Wed_ref, bed_ref, eps_ref, out_ref):
    adj = adj_ref[...]
    ef_all = ef_ref[...]
    gh = nf_ref[...]
    eps = eps_ref[...]

    dot = functools.partial(jnp.dot, preferred_element_type=jnp.float32)
    z_mu = dot(gh, Wmu_ref[...]) + bmu_ref[...]
    z_ls = dot(gh, Wls_ref[...]) + bls_ref[...]
    z = z_mu + eps * jnp.exp(0.5 * z_ls)
    y = jnp.maximum(dot(z, Wd1_ref[...]) + bd1_ref[...], 0.0)
    hdec = dot(y, Wd2_ref[...]) + bd2_ref[...]
    out = jax.nn.sigmoid(hdec)
    node_recon = dot(y, Wnd_ref[...]) + bnd_ref[...]
    ed144 = dot(y, Wed_ref[...]) + bed_ref[...]
    e_r = lax.broadcasted_iota(jnp.int32, (EM, 4 * EM), 0)
    e_c = lax.broadcasted_iota(jnp.int32, (EM, 4 * EM), 1)
    Asel = ((e_c // 4) == e_r).astype(F32)
    b_r = lax.broadcasted_iota(jnp.int32, (4 * EM, 4), 0)
    b_c = lax.broadcasted_iota(jnp.int32, (4 * EM, 4), 1)
    Bsel = ((b_r % 4) == b_c).astype(F32)
    ed_logits = dot(Asel * ed144, Bsel)

    edm = jnp.max(ed_logits, axis=1, keepdims=True)
    ede = jnp.exp(ed_logits - edm)
    er = ede / jnp.sum(ede, axis=1, keepdims=True)

    rows = []
    base = 0
    for r in range(N):
        seg = out[:, base:base + (N - r)]
        if r > 0:
            seg = jnp.concatenate([jnp.zeros((1, r), F32), seg], axis=1)
        rows.append(seg)
        base += N - r
    low = jnp.concatenate(rows, axis=0)

    r9 = lax.broadcasted_iota(jnp.int32, (N, N), 0)
    c9 = lax.broadcasted_iota(jnp.int32, (N, N), 1)
    eyeM = (r9 == c9).astype(F32)
    triuM = (r9 <= c9).astype(F32)

    def _tr(m):
        return lax.dot_general(eyeM, m, (((1,), (1,)), ((), ())),
                               preferred_element_type=jnp.float32)

    lowT = _tr(low)
    adjr = low + lowT - low * eyeM

    adjT = _tr(adj)
    aw_col = jnp.concatenate(
        [adjT[r + 1:N, r:r + 1] for r in range(N - 1)], axis=0)
    edges_total = er * aw_col

    ef9 = ef_all[:N, :]
    efr9 = er[:N, :]
    outer = functools.partial(
        lax.dot_general, dimension_numbers=(((1,), (1,)), ((), ())),
        preferred_element_type=jnp.float32)
    dots = outer(ef9, efr9)
    n1 = jnp.sqrt(jnp.sum(ef9 * ef9, axis=1, keepdims=True))
    n2 = jnp.sqrt(jnp.sum(efr9 * efr9, axis=1, keepdims=True))
    denom = jnp.maximum(outer(n1, n2), 1e-8)
    cosm = dots / denom

    dadj = jnp.sum(adj * eyeM, axis=1, keepdims=True)
    dadjr = jnp.sum(adjr * eyeM, axis=1, keepdims=True)
    diag_term = outer(dadj, dadjr) * cosm

    f_r = lax.broadcasted_iota(jnp.int32, (N * N, N), 0)
    f_c = lax.broadcasted_iota(jnp.int32, (N * N, N), 1)
    RowSel = ((f_r // N) == f_c).astype(F32)
    ModMsk = ((f_r % N) == f_c).astype(F32)
    adj_col = jnp.sum(dot(RowSel, adj) * ModMsk,
                      axis=1, keepdims=True)

    g_r = lax.broadcasted_iota(jnp.int32, (N, N * N), 0)
    g_c = lax.broadcasted_iota(jnp.int32, (N, N * N), 1)
    ColSel = (g_r == (g_c % N)).astype(F32)
    DivMsk = (g_r == (g_c // N)).astype(F32)
    adjr_row = jnp.sum(dot(adjr, ColSel) * DivMsk,
                       axis=0, keepdims=True)
    base_S = jnp.abs(adj_col - adjr_row)

    vR = lax.broadcasted_iota(jnp.int32, (N * N, 1), 0)
    vC = lax.broadcasted_iota(jnp.int32, (1, N * N), 1)
    eyeR = (vR // N) == (vR % N)
    eyeC = (vC // N) == (vC % N)
    offmask = ((~eyeR) & (~eyeC)).astype(F32)

    dt_c = jnp.concatenate([diag_term] * N, axis=1)
    dt_tile = jnp.concatenate([dt_c] * N, axis=0)
    S = jnp.where(eyeR & eyeC, dt_tile, base_S * offmask)

    rm_r = lax.broadcasted_iota(jnp.int32, (N, N * N), 0)
    rm_c = lax.broadcasted_iota(jnp.int32, (N, N * N), 1)
    Rm = (((rm_c // N) == rm_r) & ((rm_c % N) != rm_r)).astype(F32)

    def mpm_core(x):
        xcols = jnp.concatenate([x] * N, axis=0)
        pmax = jnp.concatenate(
            [jnp.max(S[:, a * N:(a + 1) * N] * xcols, axis=1, keepdims=True)
             for a in range(N)], axis=1)
        neigh = dot(Rm, pmax)
        return x * diag_term + neigh

    def mpm_outer(_, x):
        x = lax.fori_loop(0, 9, lambda __, v: mpm_core(v), x)
        x = mpm_core(x)
        return x / jnp.sqrt(jnp.sum(x * x))

    x0 = jnp.full((N, N), 1.0 / N, F32)
    x = lax.fori_loop(0, 5, mpm_outer, x0)

    iota512 = lax.broadcasted_iota(jnp.int32, (1, 512), 1)
    bitmask = [((iota512 >> j) & 1) == 1 for j in range(N)]

    g = jnp.zeros((1, 512), F32)
    AM = [None] * N
    for i in range(N - 1, -1, -1):
        m = jnp.full((1, 512), NEG, F32)
        am = jnp.zeros((1, 512), jnp.int32)
        for j in range(N):
            shifted = _roll_lanes(g, 1 << j)
            xij = x[i:i + 1, j:j + 1]
            cand = jnp.where(bitmask[j], shifted + xij, NEG)
            am = jnp.where(cand > m, j, am)
            m = jnp.maximum(m, cand)
        g = m
        AM[i] = am

    iota9r = lax.broadcasted_iota(jnp.int32, (1, N), 1)

    s_cur = jnp.int32(511)
    p_rows = []
    for i in range(N):
        j_pick = jnp.sum(jnp.where(iota512 == s_cur, AM[i], 0))
        p_rows.append((iota9r == j_pick).astype(F32))
        s_cur = s_cur - (jnp.int32(1) << j_pick)

    P = jnp.concatenate(p_rows, axis=0)
    PtA = lax.dot_general(P, adj, (((0,), (0,)), ((), ())),
                          preferred_element_type=jnp.float32)
    adj_perm = dot(PtA, P)

    pclip = jnp.clip(low, 1e-7, 1.0 - 1e-7)
    bce_mat = adj_perm * jnp.log(pclip) + (1.0 - adj_perm) * jnp.log(1.0 - pclip)
    adj_recon_loss = -jnp.sum(bce_mat * triuM) / ODIM

    loss_kl = -0.5 * jnp.sum(1.0 + z_ls - z_mu * z_mu - jnp.exp(z_ls)) / (N * N)
    diff_e = edges_total - ef_all
    loss_edge = jnp.sum(diff_e * diff_e) / (EM * 4)
    diff_n = node_recon - gh
    loss_node = jnp.sum(diff_n * diff_n) / HH

    total = adj_recon_loss + loss_kl + loss_edge + loss_node
    out_ref[...] = jnp.broadcast_to(total, (1, 1))


def kernel(adj, edges_features, nodes_features, W_mu, b_mu, W_ls, b_ls,
           W_d1, b_d1, W_d2, b_d2, W_nd, b_nd, W_ed, b_ed, eps):
    adj0 = adj[0]
    ef = edges_features[0]
    gh = nodes_features.reshape(1, HH)
    res = pl.pallas_call(
        _body,
        out_shape=jax.ShapeDtypeStruct((1, 1), jnp.float32),
    )(adj0, ef, gh,
      W_mu, b_mu.reshape(1, -1), W_ls, b_ls.reshape(1, -1),
      W_d1, b_d1.reshape(1, -1), W_d2, b_d2.reshape(1, -1),
      W_nd, b_nd.reshape(1, -1), W_ed, b_ed.reshape(1, -1),
      eps.reshape(1, -1))
    return res[0, 0]
